# Initial kernel scaffold; baseline (speedup 1.0000x reference)
import jax
import jax.numpy as jnp
from jax.experimental import pallas as pl


def kernel(node_feats, adj_matrix, W, b, a):
    raise NotImplementedError("write your pallas kernel here")


# speedup vs baseline: 11.3308x; 11.3308x over previous
import functools

import jax
import jax.numpy as jnp
from jax.experimental import pallas as pl
from jax.experimental.pallas import tpu as pltpu

_NEG = -9e15
_ALPHA = 0.2


def _gat_kernel(nf_ref, adj_ref, w_ref, b_ref, asrc_ref, adst_ref, out_ref,
                nfp_ref, s_ref, t_ref, *, num_heads, c_head, blk):
    i = pl.program_id(0)
    hc = num_heads * c_head

    @pl.when(i == 0)
    def _init():
        nfp = jax.lax.dot_general(
            nf_ref[...], w_ref[...],
            (((1,), (1,)), ((), ())),
            preferred_element_type=jnp.float32) + b_ref[...]
        nfp_ref[...] = nfp
        row = jax.lax.broadcasted_iota(jnp.int32, (hc, num_heads), 0) // c_head
        col = jax.lax.broadcasted_iota(jnp.int32, (hc, num_heads), 1)
        mask = (row == col).astype(jnp.float32)
        a_src = asrc_ref[...] * mask
        a_dst = adst_ref[...] * mask
        s_ref[...] = jnp.dot(nfp, a_src, preferred_element_type=jnp.float32)
        t_ref[...] = jax.lax.dot_general(
            a_dst, nfp, (((0,), (1,)), ((), ())),
            preferred_element_type=jnp.float32)

    nfp = nfp_ref[...]
    adj_ok = adj_ref[...] != 0
    for h in range(num_heads):
        s_h = s_ref[pl.ds(i * blk, blk), h:h + 1]
        t_h = t_ref[h:h + 1, :]
        logits = s_h + t_h
        logits = jnp.where(logits >= 0, logits, _ALPHA * logits)
        masked = jnp.where(adj_ok, logits, _NEG)
        mx = jnp.max(masked, axis=1, keepdims=True)
        e = jnp.exp(masked - mx)
        den = jnp.sum(e, axis=1, keepdims=True)
        probs = e / den
        out_ref[:, h * c_head:(h + 1) * c_head] = jnp.dot(
            probs, nfp[:, h * c_head:(h + 1) * c_head],
            preferred_element_type=jnp.float32)


def kernel(node_feats, adj_matrix, W, b, a):
    B, N, c_in = node_feats.shape
    num_heads = a.shape[0]
    c_head = a.shape[1] // 2
    hc = num_heads * c_head

    nf = node_feats.reshape(N, c_in)
    adj = adj_matrix.reshape(N, N)
    a_src = a[:, :c_head].reshape(hc, 1)
    a_dst = a[:, c_head:].reshape(hc, 1)
    b2 = b.reshape(1, hc)

    blk = 256
    out = pl.pallas_call(
        functools.partial(_gat_kernel, num_heads=num_heads, c_head=c_head,
                          blk=blk),
        grid=(N // blk,),
        in_specs=[
            pl.BlockSpec((N, c_in), lambda i: (0, 0)),
            pl.BlockSpec((blk, N), lambda i: (i, 0)),
            pl.BlockSpec((hc, c_in), lambda i: (0, 0)),
            pl.BlockSpec((1, hc), lambda i: (0, 0)),
            pl.BlockSpec((hc, 1), lambda i: (0, 0)),
            pl.BlockSpec((hc, 1), lambda i: (0, 0)),
        ],
        out_specs=pl.BlockSpec((blk, hc), lambda i: (i, 0)),
        out_shape=jax.ShapeDtypeStruct((N, hc), jnp.float32),
        scratch_shapes=[
            pltpu.VMEM((N, hc), jnp.float32),
            pltpu.VMEM((N, num_heads), jnp.float32),
            pltpu.VMEM((num_heads, N), jnp.float32),
        ],
    )(nf, adj, W, b2, a_src, a_dst)
    return out.reshape(B, N, hc)
